# baseline (speedup 1.0000x reference)
import dataclasses
import functools

import jax
import jax.numpy as jnp
from jax import lax
from jax.experimental import pallas as pl
from jax.experimental.pallas import tpu as pltpu
from jax.experimental.pallas import tpu_sc as plsc

N_NODES = 10000
D = 128
NC, NS = 2, 16
NW = NC * NS
N_PAD = 10240
CPT = 80
E_PAD = NW * CPT * 128
ROWS_PER_TILE = N_PAD // NS


def _sc_compiler_params():
    cp = pltpu.CompilerParams()
    if "needs_layout_passes" in pltpu.CompilerParams.__dataclass_fields__:
        cp = dataclasses.replace(cp, needs_layout_passes=False)
    return cp


def _sc_aggregate(feat, src2d, dst2d):
    mesh = plsc.VectorSubcoreMesh(core_axis_name="c", subcore_axis_name="s")

    @functools.partial(
        pl.kernel,
        mesh=mesh,
        compiler_params=_sc_compiler_params(),
        out_type=(
            jax.ShapeDtypeStruct((NC, N_PAD, D), jnp.float32),
            jax.ShapeDtypeStruct((NC, NS, N_PAD), jnp.float32),
        ),
        scratch_types=[
            pltpu.VMEM((CPT, 128), jnp.int32),
            pltpu.VMEM((CPT, 128), jnp.int32),
            pltpu.VMEM((128, D), jnp.float32),
            pltpu.VMEM((N_PAD,), jnp.float32),
            pltpu.VMEM_SHARED((N_PAD, D), jnp.float32),
            pltpu.SemaphoreType.DMA,
        ],
    )
    def agg(feat_h, src_h, dst_h, p_out, deg_out,
            src_v, dst_v, rows_v, deg_v, acc_sh, g0):
        c = lax.axis_index("c")
        s = lax.axis_index("s")
        zeros16 = jnp.zeros((16,), jnp.float32)
        ones16 = jnp.ones((16,), jnp.float32)

        @pl.loop(0, N_PAD // 16)
        def _(i):
            deg_v[pl.ds(i * 16, 16)] = zeros16

        @pl.loop(0, 128)
        def _(r):
            @pl.loop(0, D // 16)
            def _(k):
                rows_v[r, pl.ds(k * 16, 16)] = zeros16

        nbase = s * ROWS_PER_TILE

        @pl.loop(0, ROWS_PER_TILE // 128)
        def _(i):
            pltpu.sync_copy(rows_v, acc_sh.at[pl.ds(nbase + i * 128, 128)])

        plsc.subcore_barrier()

        ebase = (c * NS + s) * CPT
        pltpu.sync_copy(src_h.at[pl.ds(ebase, CPT)], src_v)
        pltpu.sync_copy(dst_h.at[pl.ds(ebase, CPT)], dst_v)

        def _deg_update(j):
            @pl.loop(0, 128 // 16)
            def _(k):
                idx16 = dst_v[j, pl.ds(k * 16, 16)]
                plsc.addupdate_scatter(deg_v, [idx16], ones16)

        @pl.loop(0, CPT)
        def _(j):
            pltpu.async_copy(feat_h.at[pl.ds(1024, 128)], rows_v, g0).wait()
            _deg_update(j)

        plsc.subcore_barrier()

        pltpu.sync_copy(acc_sh.at[pl.ds(nbase, ROWS_PER_TILE)],
                        p_out.at[c, pl.ds(nbase, ROWS_PER_TILE)])
        pltpu.sync_copy(deg_v, deg_out.at[c, s])

    return agg(feat, src2d, dst2d)


def _tc_linear(feat, p0, p1, deg_t, W_self, b_self, W_neigh, b_neigh):
    blk = 400
    dn = (((1,), (1,)), ((), ()))

    def body(feat_b, p0_b, p1_b, deg_b, ws_b, bs_b, wn_b, bn_b, out_b):
        hsum = p0_b[...] + p1_b[...]
        deg = jnp.sum(deg_b[...], axis=1, keepdims=True)
        hn = jnp.where(deg > 0.0, hsum / jnp.maximum(deg, 1.0), 0.0)
        out_b[...] = (
            lax.dot_general(feat_b[...], ws_b[...], dn,
                            preferred_element_type=jnp.float32)
            + lax.dot_general(hn, wn_b[...], dn,
                              preferred_element_type=jnp.float32)
            + bs_b[...] + bn_b[...]
        )

    return pl.pallas_call(
        body,
        grid=(N_NODES // blk,),
        in_specs=[
            pl.BlockSpec((blk, D), lambda i: (i, 0)),
            pl.BlockSpec((blk, D), lambda i: (i, 0)),
            pl.BlockSpec((blk, D), lambda i: (i, 0)),
            pl.BlockSpec((blk, NW), lambda i: (i, 0)),
            pl.BlockSpec((D, D), lambda i: (0, 0)),
            pl.BlockSpec((1, D), lambda i: (0, 0)),
            pl.BlockSpec((D, D), lambda i: (0, 0)),
            pl.BlockSpec((1, D), lambda i: (0, 0)),
        ],
        out_specs=pl.BlockSpec((blk, D), lambda i: (i, 0)),
        out_shape=jax.ShapeDtypeStruct((N_NODES, D), jnp.float32),
    )(feat, p0, p1, deg_t, W_self, b_self.reshape(1, D),
      W_neigh, b_neigh.reshape(1, D))


def kernel(feat, edge_index, W_self, b_self, W_neigh, b_neigh):
    src = edge_index[0]
    dst = edge_index[1]
    pad = E_PAD - src.shape[0]
    def _to_tile_chunks(x):
        return x.reshape(CPT, NW, 128).transpose(1, 0, 2).reshape(E_PAD // 128, 128)

    src_p = _to_tile_chunks(jnp.concatenate([src, jnp.zeros((pad,), jnp.int32)]))
    pad_dst = N_NODES + jnp.arange(pad, dtype=jnp.int32) % (N_PAD - N_NODES)
    dst_p = _to_tile_chunks(jnp.concatenate([dst, pad_dst]))
    p, degp = _sc_aggregate(feat, src_p, dst_p)
    deg_t = degp.reshape(NW, N_PAD).transpose(1, 0)
    return _tc_linear(feat, p[0], p[1], deg_t,
                      W_self, b_self, W_neigh, b_neigh)


# speedup vs baseline: 1.0115x; 1.0115x over previous
import dataclasses
import functools

import jax
import jax.numpy as jnp
from jax import lax
from jax.experimental import pallas as pl
from jax.experimental.pallas import tpu as pltpu
from jax.experimental.pallas import tpu_sc as plsc

N_NODES = 10000
D = 128
DH = D // 2
NC, NS = 2, 16
NW = NC * NS
N_PAD = 10240
CPT = 160
E_PAD = NS * CPT * 128
ROWS_PER_TILE = N_PAD // NS


def _sc_compiler_params():
    cp = pltpu.CompilerParams()
    fields = pltpu.CompilerParams.__dataclass_fields__
    if "needs_layout_passes" in fields:
        cp = dataclasses.replace(cp, needs_layout_passes=False)
    if "use_tc_tiling_on_sc" in fields:
        cp = dataclasses.replace(cp, use_tc_tiling_on_sc=False)
    return cp


def _sc_aggregate(feat2, src2d, dst2d):
    mesh = plsc.VectorSubcoreMesh(core_axis_name="c", subcore_axis_name="s")

    @functools.partial(
        pl.kernel,
        mesh=mesh,
        compiler_params=_sc_compiler_params(),
        out_type=(
            jax.ShapeDtypeStruct((NC, N_PAD, DH), jnp.float32),
            jax.ShapeDtypeStruct((NC, NS, N_PAD), jnp.float32),
        ),
        scratch_types=[
            pltpu.VMEM((CPT, 128), jnp.int32),
            pltpu.VMEM((CPT, 128), jnp.int32),
            pltpu.VMEM((128, DH), jnp.float32),
            pltpu.VMEM((128, DH), jnp.float32),
            pltpu.VMEM((N_PAD,), jnp.float32),
            pltpu.VMEM_SHARED((N_PAD, DH), jnp.float32),
            pltpu.SemaphoreType.DMA,
            pltpu.SemaphoreType.DMA,
        ],
    )
    def agg(feat_h, src_h, dst_h, p_out, deg_out,
            src_v, dst_v, rows_a, rows_b, deg_v, acc_sh, ga, gb):
        c = lax.axis_index("c")
        s = lax.axis_index("s")
        zeros16 = jnp.zeros((16,), jnp.float32)
        ones16 = jnp.ones((16,), jnp.float32)

        @pl.loop(0, N_PAD // 16)
        def _(i):
            deg_v[pl.ds(i * 16, 16)] = zeros16

        @pl.loop(0, 128)
        def _(r):
            @pl.loop(0, DH // 16)
            def _(k):
                rows_a[r, pl.ds(k * 16, 16)] = zeros16

        nbase = s * ROWS_PER_TILE

        @pl.loop(0, ROWS_PER_TILE // 128)
        def _(i):
            pltpu.sync_copy(rows_a, acc_sh.at[pl.ds(nbase + i * 128, 128)])

        plsc.subcore_barrier()

        ebase = s * CPT
        pltpu.sync_copy(src_h.at[pl.ds(ebase, CPT)], src_v)
        pltpu.sync_copy(dst_h.at[pl.ds(ebase, CPT)], dst_v)

        def _deg_update(j):
            @pl.when(jnp.where(c == 0, j < CPT // 2, j >= CPT // 2))
            def _():
                @pl.loop(0, 128 // 16)
                def _(k):
                    idx16 = dst_v[j, pl.ds(k * 16, 16)]
                    plsc.addupdate_scatter(deg_v, [idx16], ones16)

        fb = feat_h.at[c]

        pltpu.async_copy(fb.at[src_v.at[0]], rows_a, ga)

        @pl.loop(0, CPT // 2)
        def _(g):
            j0 = 2 * g
            j1 = j0 + 1
            pltpu.async_copy(fb.at[src_v.at[j1]], rows_b, gb)
            pltpu.make_async_copy(fb.at[src_v.at[j0]], rows_a, ga).wait()
            pltpu.sync_copy(rows_a, acc_sh.at[dst_v.at[j0]], add=True)
            _deg_update(j0)

            @pl.when(g < CPT // 2 - 1)
            def _():
                pltpu.async_copy(fb.at[src_v.at[j0 + 2]], rows_a, ga)

            pltpu.make_async_copy(fb.at[src_v.at[j1]], rows_b, gb).wait()
            pltpu.sync_copy(rows_b, acc_sh.at[dst_v.at[j1]], add=True)
            _deg_update(j1)

        plsc.subcore_barrier()

        pltpu.sync_copy(acc_sh.at[pl.ds(nbase, ROWS_PER_TILE)],
                        p_out.at[c, pl.ds(nbase, ROWS_PER_TILE)])
        pltpu.sync_copy(deg_v, deg_out.at[c, s])

    return agg(feat2, src2d, dst2d)


def _tc_linear(feat, p0, p1, deg_t, W_self, Wn0, Wn1, b_self, b_neigh):
    blk = 400
    dn = (((1,), (1,)), ((), ()))

    def body(feat_b, p0_b, p1_b, deg_b, ws_b, wn0_b, wn1_b, bs_b, bn_b, out_b):
        deg = jnp.sum(deg_b[...], axis=1, keepdims=True)
        scale = jnp.where(deg > 0.0, 1.0 / jnp.maximum(deg, 1.0), 0.0)
        h0 = p0_b[...] * scale
        h1 = p1_b[...] * scale
        out_b[...] = (
            lax.dot_general(feat_b[...], ws_b[...], dn,
                            preferred_element_type=jnp.float32)
            + lax.dot_general(h0, wn0_b[...], dn,
                              preferred_element_type=jnp.float32)
            + lax.dot_general(h1, wn1_b[...], dn,
                              preferred_element_type=jnp.float32)
            + bs_b[...] + bn_b[...]
        )

    return pl.pallas_call(
        body,
        grid=(N_NODES // blk,),
        in_specs=[
            pl.BlockSpec((blk, D), lambda i: (i, 0)),
            pl.BlockSpec((blk, DH), lambda i: (i, 0)),
            pl.BlockSpec((blk, DH), lambda i: (i, 0)),
            pl.BlockSpec((blk, NW), lambda i: (i, 0)),
            pl.BlockSpec((D, D), lambda i: (0, 0)),
            pl.BlockSpec((D, DH), lambda i: (0, 0)),
            pl.BlockSpec((D, DH), lambda i: (0, 0)),
            pl.BlockSpec((1, D), lambda i: (0, 0)),
            pl.BlockSpec((1, D), lambda i: (0, 0)),
        ],
        out_specs=pl.BlockSpec((blk, D), lambda i: (i, 0)),
        out_shape=jax.ShapeDtypeStruct((N_NODES, D), jnp.float32),
    )(feat, p0, p1, deg_t, W_self, Wn0, Wn1,
      b_self.reshape(1, D), b_neigh.reshape(1, D))


def kernel(feat, edge_index, W_self, b_self, W_neigh, b_neigh):
    src = edge_index[0]
    dst = edge_index[1]
    pad = E_PAD - src.shape[0]

    def _to_tile_chunks(x):
        return x.reshape(CPT, NS, 128).transpose(1, 0, 2).reshape(E_PAD // 128, 128)

    src_p = _to_tile_chunks(jnp.concatenate([src, jnp.zeros((pad,), jnp.int32)]))
    pad_dst = N_NODES + jnp.arange(pad, dtype=jnp.int32) % (N_PAD - N_NODES)
    dst_p = _to_tile_chunks(jnp.concatenate([dst, pad_dst]))

    feat2 = jnp.stack([feat[:, :DH], feat[:, DH:]])
    p, degp = _sc_aggregate(feat2, src_p, dst_p)
    deg_t = degp.reshape(NW, N_PAD).transpose(1, 0)
    return _tc_linear(feat, p[0], p[1], deg_t,
                      W_self, W_neigh[:, :DH], W_neigh[:, DH:],
                      b_self, b_neigh)


# speedup vs baseline: 1.0528x; 1.0409x over previous
import dataclasses
import functools

import jax
import jax.numpy as jnp
from jax import lax
from jax.experimental import pallas as pl
from jax.experimental.pallas import tpu as pltpu
from jax.experimental.pallas import tpu_sc as plsc

N_NODES = 10000
D = 128
DH = D // 2
NC, NS = 2, 16
NW = NC * NS
N_PAD = 10240
CPT = 160
E_PAD = NS * CPT * 128
ROWS_PER_TILE = N_PAD // NS


def _sc_compiler_params():
    cp = pltpu.CompilerParams()
    fields = pltpu.CompilerParams.__dataclass_fields__
    if "needs_layout_passes" in fields:
        cp = dataclasses.replace(cp, needs_layout_passes=False)
    if "use_tc_tiling_on_sc" in fields:
        cp = dataclasses.replace(cp, use_tc_tiling_on_sc=False)
    return cp


def _sc_aggregate(feat2, src2d, dst2d):
    mesh = plsc.VectorSubcoreMesh(core_axis_name="c", subcore_axis_name="s")

    @functools.partial(
        pl.kernel,
        mesh=mesh,
        compiler_params=_sc_compiler_params(),
        out_type=(
            jax.ShapeDtypeStruct((NC, N_PAD, DH), jnp.float32),
            jax.ShapeDtypeStruct((NC, NS, N_PAD), jnp.float32),
        ),
        scratch_types=[
            pltpu.VMEM((CPT, 128), jnp.int32),
            pltpu.VMEM((CPT, 128), jnp.int32),
            pltpu.VMEM((128, DH), jnp.float32),
            pltpu.VMEM((128, DH), jnp.float32),
            pltpu.VMEM((128, DH), jnp.float32),
            pltpu.VMEM((128, DH), jnp.float32),
            pltpu.VMEM((N_PAD,), jnp.float32),
            pltpu.VMEM_SHARED((N_PAD, DH), jnp.float32),
            pltpu.SemaphoreType.DMA,
            pltpu.SemaphoreType.DMA,
            pltpu.SemaphoreType.DMA,
            pltpu.SemaphoreType.DMA,
            pltpu.SemaphoreType.DMA,
            pltpu.SemaphoreType.DMA,
            pltpu.SemaphoreType.DMA,
            pltpu.SemaphoreType.DMA,
        ],
    )
    def agg(feat_h, src_h, dst_h, p_out, deg_out,
            src_v, dst_v, r0, r1, r2, r3, deg_v, acc_sh,
            g0, g1, g2, g3, t0, t1, t2, t3):
        c = lax.axis_index("c")
        s = lax.axis_index("s")
        zeros16 = jnp.zeros((16,), jnp.float32)
        ones16 = jnp.ones((16,), jnp.float32)

        @pl.loop(0, N_PAD // 16)
        def _(i):
            deg_v[pl.ds(i * 16, 16)] = zeros16

        @pl.loop(0, 128)
        def _(r):
            @pl.loop(0, DH // 16)
            def _(k):
                r0[r, pl.ds(k * 16, 16)] = zeros16

        nbase = s * ROWS_PER_TILE

        @pl.loop(0, ROWS_PER_TILE // 128)
        def _(i):
            pltpu.sync_copy(r0, acc_sh.at[pl.ds(nbase + i * 128, 128)])

        plsc.subcore_barrier()

        ebase = s * CPT
        pltpu.sync_copy(src_h.at[pl.ds(ebase, CPT)], src_v)
        pltpu.sync_copy(dst_h.at[pl.ds(ebase, CPT)], dst_v)

        def _deg_update(j):
            @pl.when(jnp.where(c == 0, j < CPT // 2, j >= CPT // 2))
            def _():
                @pl.loop(0, 128 // 16)
                def _(k):
                    idx16 = dst_v[j, pl.ds(k * 16, 16)]
                    plsc.addupdate_scatter(deg_v, [idx16], ones16)

        fb = feat_h.at[c]
        bufs = (r0, r1, r2, r3)
        gsem = (g0, g1, g2, g3)
        tsem = (t0, t1, t2, t3)
        NBUF = 4

        for b in range(NBUF):
            pltpu.async_copy(fb.at[src_v.at[b]], bufs[b], gsem[b])

        @pl.loop(0, CPT // NBUF)
        def _(q):
            base = q * NBUF
            for b in range(NBUF):
                j = base + b
                pltpu.make_async_copy(fb.at[src_v.at[j]], bufs[b], gsem[b]).wait()
                pltpu.async_copy(bufs[b], acc_sh.at[dst_v.at[j]], tsem[b],
                                 add=True)
                _deg_update(j)
            for b in range(NBUF):
                j = base + b

                @pl.when(j + NBUF < CPT)
                def _():
                    pltpu.make_async_copy(bufs[b], acc_sh.at[dst_v.at[j]],
                                          tsem[b]).wait()
                    pltpu.async_copy(fb.at[src_v.at[j + NBUF]], bufs[b], gsem[b])

        for b in range(NBUF):
            j = CPT - NBUF + b
            pltpu.make_async_copy(bufs[b], acc_sh.at[dst_v.at[j]],
                                  tsem[b]).wait()

        plsc.subcore_barrier()

        pltpu.sync_copy(acc_sh.at[pl.ds(nbase, ROWS_PER_TILE)],
                        p_out.at[c, pl.ds(nbase, ROWS_PER_TILE)])
        pltpu.sync_copy(deg_v, deg_out.at[c, s])

    return agg(feat2, src2d, dst2d)


def _tc_linear(feat, p0, p1, deg_t, W_self, Wn0, Wn1, b_self, b_neigh):
    blk = 400
    dn = (((1,), (1,)), ((), ()))

    def body(feat_b, p0_b, p1_b, deg_b, ws_b, wn0_b, wn1_b, bs_b, bn_b, out_b):
        deg = jnp.sum(deg_b[...], axis=1, keepdims=True)
        scale = jnp.where(deg > 0.0, 1.0 / jnp.maximum(deg, 1.0), 0.0)
        h0 = p0_b[...] * scale
        h1 = p1_b[...] * scale
        out_b[...] = (
            lax.dot_general(feat_b[...], ws_b[...], dn,
                            preferred_element_type=jnp.float32)
            + lax.dot_general(h0, wn0_b[...], dn,
                              preferred_element_type=jnp.float32)
            + lax.dot_general(h1, wn1_b[...], dn,
                              preferred_element_type=jnp.float32)
            + bs_b[...] + bn_b[...]
        )

    return pl.pallas_call(
        body,
        grid=(N_NODES // blk,),
        in_specs=[
            pl.BlockSpec((blk, D), lambda i: (i, 0)),
            pl.BlockSpec((blk, DH), lambda i: (i, 0)),
            pl.BlockSpec((blk, DH), lambda i: (i, 0)),
            pl.BlockSpec((blk, NW), lambda i: (i, 0)),
            pl.BlockSpec((D, D), lambda i: (0, 0)),
            pl.BlockSpec((D, DH), lambda i: (0, 0)),
            pl.BlockSpec((D, DH), lambda i: (0, 0)),
            pl.BlockSpec((1, D), lambda i: (0, 0)),
            pl.BlockSpec((1, D), lambda i: (0, 0)),
        ],
        out_specs=pl.BlockSpec((blk, D), lambda i: (i, 0)),
        out_shape=jax.ShapeDtypeStruct((N_NODES, D), jnp.float32),
    )(feat, p0, p1, deg_t, W_self, Wn0, Wn1,
      b_self.reshape(1, D), b_neigh.reshape(1, D))


def kernel(feat, edge_index, W_self, b_self, W_neigh, b_neigh):
    src = edge_index[0]
    dst = edge_index[1]
    pad = E_PAD - src.shape[0]

    def _to_tile_chunks(x):
        return x.reshape(CPT, NS, 128).transpose(1, 0, 2).reshape(E_PAD // 128, 128)

    src_p = _to_tile_chunks(jnp.concatenate([src, jnp.zeros((pad,), jnp.int32)]))
    pad_dst = N_NODES + jnp.arange(pad, dtype=jnp.int32) % (N_PAD - N_NODES)
    dst_p = _to_tile_chunks(jnp.concatenate([dst, pad_dst]))

    feat2 = jnp.stack([feat[:, :DH], feat[:, DH:]])
    p, degp = _sc_aggregate(feat2, src_p, dst_p)
    deg_t = degp.reshape(NW, N_PAD).transpose(1, 0)
    return _tc_linear(feat, p[0], p[1], deg_t,
                      W_self, W_neigh[:, :DH], W_neigh[:, DH:],
                      b_self, b_neigh)


# speedup vs baseline: 1.0863x; 1.0318x over previous
import dataclasses
import functools

import jax
import jax.numpy as jnp
from jax import lax
from jax.experimental import pallas as pl
from jax.experimental.pallas import tpu as pltpu
from jax.experimental.pallas import tpu_sc as plsc

N_NODES = 10000
D = 128
DH = D // 2
NC, NS = 2, 16
NW = NC * NS
N_PAD = 10240
CPT = 160
E_PAD = NS * CPT * 128
ROWS_PER_TILE = N_PAD // NS


def _sc_compiler_params():
    cp = pltpu.CompilerParams()
    fields = pltpu.CompilerParams.__dataclass_fields__
    if "needs_layout_passes" in fields:
        cp = dataclasses.replace(cp, needs_layout_passes=False)
    if "use_tc_tiling_on_sc" in fields:
        cp = dataclasses.replace(cp, use_tc_tiling_on_sc=False)
    return cp


def _sc_aggregate(feat2, src2d, dst2d):
    mesh = plsc.VectorSubcoreMesh(core_axis_name="c", subcore_axis_name="s")

    @functools.partial(
        pl.kernel,
        mesh=mesh,
        compiler_params=_sc_compiler_params(),
        out_type=(
            jax.ShapeDtypeStruct((NC, N_PAD, DH), jnp.float32),
            jax.ShapeDtypeStruct((NC, NS, N_PAD), jnp.float32),
        ),
        scratch_types=[
            pltpu.VMEM((CPT, 128), jnp.int32),
            pltpu.VMEM((CPT, 128), jnp.int32),
            pltpu.VMEM((128, DH), jnp.float32),
            pltpu.VMEM((128, DH), jnp.float32),
            pltpu.VMEM((128, DH), jnp.float32),
            pltpu.VMEM((128, DH), jnp.float32),
            pltpu.VMEM((N_PAD,), jnp.float32),
            pltpu.VMEM_SHARED((N_PAD, DH), jnp.float32),
            pltpu.SemaphoreType.DMA,
            pltpu.SemaphoreType.DMA,
            pltpu.SemaphoreType.DMA,
            pltpu.SemaphoreType.DMA,
            pltpu.SemaphoreType.DMA,
            pltpu.SemaphoreType.DMA,
            pltpu.SemaphoreType.DMA,
            pltpu.SemaphoreType.DMA,
        ],
    )
    def agg(feat_h, src_h, dst_h, p_out, deg_out,
            src_v, dst_v, r0, r1, r2, r3, deg_v, acc_sh,
            g0, g1, g2, g3, t0, t1, t2, t3):
        c = lax.axis_index("c")
        s = lax.axis_index("s")
        zeros16 = jnp.zeros((16,), jnp.float32)
        ones16 = jnp.ones((16,), jnp.float32)

        @pl.loop(0, N_PAD // 16)
        def _(i):
            deg_v[pl.ds(i * 16, 16)] = zeros16

        @pl.loop(0, 128)
        def _(r):
            @pl.loop(0, DH // 16)
            def _(k):
                r0[r, pl.ds(k * 16, 16)] = zeros16

        nbase = s * ROWS_PER_TILE

        @pl.loop(0, ROWS_PER_TILE // 128)
        def _(i):
            pltpu.sync_copy(r0, acc_sh.at[pl.ds(nbase + i * 128, 128)])

        plsc.subcore_barrier()

        ebase = s * CPT
        pltpu.sync_copy(src_h.at[pl.ds(ebase, CPT)], src_v)
        pltpu.sync_copy(dst_h.at[pl.ds(ebase, CPT)], dst_v)

        def _deg_update(j):
            @pl.when(jnp.where(c == 0, j < CPT // 2, j >= CPT // 2))
            def _():
                @pl.loop(0, 128 // 16)
                def _(k):
                    idx16 = dst_v[j, pl.ds(k * 16, 16)]
                    plsc.addupdate_scatter(deg_v, [idx16], ones16)

        fb = feat_h.at[c]
        bufs = (r0, r1, r2, r3)
        gsem = (g0, g1, g2, g3)
        tsem = (t0, t1, t2, t3)
        NBUF = 4

        for b in range(NBUF):
            pltpu.async_copy(fb.at[src_v.at[b]], bufs[b], gsem[b])

        @pl.loop(0, CPT // NBUF)
        def _(q):
            base = q * NBUF
            for b in range(NBUF):
                j = base + b
                pltpu.make_async_copy(fb.at[src_v.at[j]], bufs[b], gsem[b]).wait()
                pltpu.async_copy(bufs[b], acc_sh.at[dst_v.at[j]], tsem[b],
                                 add=True)
                _deg_update(j)
            for b in range(NBUF):
                j = base + b

                @pl.when(j + NBUF < CPT)
                def _():
                    pltpu.make_async_copy(bufs[b], acc_sh.at[dst_v.at[j]],
                                          tsem[b]).wait()
                    pltpu.async_copy(fb.at[src_v.at[j + NBUF]], bufs[b], gsem[b])

        for b in range(NBUF):
            j = CPT - NBUF + b
            pltpu.make_async_copy(bufs[b], acc_sh.at[dst_v.at[j]],
                                  tsem[b]).wait()

        plsc.subcore_barrier()

        pltpu.sync_copy(acc_sh.at[pl.ds(nbase, ROWS_PER_TILE)],
                        p_out.at[c, pl.ds(nbase, ROWS_PER_TILE)])
        pltpu.sync_copy(deg_v, deg_out.at[c, s])

    return agg(feat2, src2d, dst2d)


def _tc_linear(feat, p0, p1, deg_t, W_self, Wn0, Wn1, b_self, b_neigh):
    blk = 2000
    dn = (((1,), (1,)), ((), ()))

    def body(feat_b, p0_b, p1_b, deg_b, ws_b, wn0_b, wn1_b, bs_b, bn_b, out_b):
        deg = jnp.sum(deg_b[...], axis=1, keepdims=True)
        scale = jnp.where(deg > 0.0, 1.0 / jnp.maximum(deg, 1.0), 0.0)
        h0 = p0_b[...] * scale
        h1 = p1_b[...] * scale
        out_b[...] = (
            lax.dot_general(feat_b[...], ws_b[...], dn,
                            preferred_element_type=jnp.float32)
            + lax.dot_general(h0, wn0_b[...], dn,
                              preferred_element_type=jnp.float32)
            + lax.dot_general(h1, wn1_b[...], dn,
                              preferred_element_type=jnp.float32)
            + bs_b[...] + bn_b[...]
        )

    return pl.pallas_call(
        body,
        grid=(N_NODES // blk,),
        in_specs=[
            pl.BlockSpec((blk, D), lambda i: (i, 0)),
            pl.BlockSpec((blk, DH), lambda i: (i, 0)),
            pl.BlockSpec((blk, DH), lambda i: (i, 0)),
            pl.BlockSpec((blk, NW), lambda i: (i, 0)),
            pl.BlockSpec((D, D), lambda i: (0, 0)),
            pl.BlockSpec((D, DH), lambda i: (0, 0)),
            pl.BlockSpec((D, DH), lambda i: (0, 0)),
            pl.BlockSpec((1, D), lambda i: (0, 0)),
            pl.BlockSpec((1, D), lambda i: (0, 0)),
        ],
        out_specs=pl.BlockSpec((blk, D), lambda i: (i, 0)),
        out_shape=jax.ShapeDtypeStruct((N_NODES, D), jnp.float32),
    )(feat, p0, p1, deg_t, W_self, Wn0, Wn1,
      b_self.reshape(1, D), b_neigh.reshape(1, D))


def kernel(feat, edge_index, W_self, b_self, W_neigh, b_neigh):
    src = edge_index[0]
    dst = edge_index[1]
    pad = E_PAD - src.shape[0]

    def _to_tile_chunks(x):
        return x.reshape(CPT, NS, 128).transpose(1, 0, 2).reshape(E_PAD // 128, 128)

    src_p = _to_tile_chunks(jnp.concatenate([src, jnp.zeros((pad,), jnp.int32)]))
    pad_dst = N_NODES + jnp.arange(pad, dtype=jnp.int32) % (N_PAD - N_NODES)
    dst_p = _to_tile_chunks(jnp.concatenate([dst, pad_dst]))

    feat2 = jnp.stack([feat[:, :DH], feat[:, DH:]])
    p, degp = _sc_aggregate(feat2, src_p, dst_p)
    deg_t = degp.reshape(NW, N_PAD).transpose(1, 0)
    return _tc_linear(feat, p[0], p[1], deg_t,
                      W_self, W_neigh[:, :DH], W_neigh[:, DH:],
                      b_self, b_neigh)


# speedup vs baseline: 1.0957x; 1.0086x over previous
import dataclasses
import functools

import jax
import jax.numpy as jnp
from jax import lax
from jax.experimental import pallas as pl
from jax.experimental.pallas import tpu as pltpu
from jax.experimental.pallas import tpu_sc as plsc

N_NODES = 10000
D = 128
DH = D // 2
NC, NS = 2, 16
NW = NC * NS
N_PAD = 10240
CPT = 160
E_PAD = NS * CPT * 128
ROWS_PER_TILE = N_PAD // NS


def _sc_compiler_params():
    cp = pltpu.CompilerParams()
    fields = pltpu.CompilerParams.__dataclass_fields__
    if "needs_layout_passes" in fields:
        cp = dataclasses.replace(cp, needs_layout_passes=False)
    if "use_tc_tiling_on_sc" in fields:
        cp = dataclasses.replace(cp, use_tc_tiling_on_sc=False)
    return cp


def _sc_aggregate(feat2, src2d, dst2d):
    mesh = plsc.VectorSubcoreMesh(core_axis_name="c", subcore_axis_name="s")

    @functools.partial(
        pl.kernel,
        mesh=mesh,
        compiler_params=_sc_compiler_params(),
        out_type=(
            jax.ShapeDtypeStruct((NC, N_PAD, DH), jnp.float32),
            jax.ShapeDtypeStruct((NC, NS, N_PAD), jnp.float32),
        ),
        scratch_types=[
            pltpu.VMEM((CPT, 128), jnp.int32),
            pltpu.VMEM((CPT, 128), jnp.int32),
            pltpu.VMEM((128, DH), jnp.float32),
            pltpu.VMEM((128, DH), jnp.float32),
            pltpu.VMEM((128, DH), jnp.float32),
            pltpu.VMEM((128, DH), jnp.float32),
            pltpu.VMEM((N_PAD,), jnp.float32),
            pltpu.VMEM_SHARED((N_PAD, DH), jnp.float32),
            pltpu.SemaphoreType.DMA,
            pltpu.SemaphoreType.DMA,
            pltpu.SemaphoreType.DMA,
            pltpu.SemaphoreType.DMA,
            pltpu.SemaphoreType.DMA,
            pltpu.SemaphoreType.DMA,
            pltpu.SemaphoreType.DMA,
            pltpu.SemaphoreType.DMA,
        ],
    )
    def agg(feat_h, src_h, dst_h, p_out, deg_out,
            src_v, dst_v, r0, r1, r2, r3, deg_v, acc_sh,
            g0, g1, g2, g3, t0, t1, t2, t3):
        c = lax.axis_index("c")
        s = lax.axis_index("s")
        zeros16 = jnp.zeros((16,), jnp.float32)
        ones16 = jnp.ones((16,), jnp.float32)

        ebase = s * CPT
        pltpu.async_copy(src_h.at[pl.ds(ebase, CPT)], src_v, g0)
        pltpu.async_copy(dst_h.at[pl.ds(ebase, CPT)], dst_v, g1)

        @pl.loop(0, N_PAD // 16)
        def _(i):
            deg_v[pl.ds(i * 16, 16)] = zeros16

        @pl.loop(0, 128)
        def _(r):
            @pl.loop(0, DH // 16)
            def _(k):
                r0[r, pl.ds(k * 16, 16)] = zeros16

        nbase = s * ROWS_PER_TILE

        @pl.loop(0, ROWS_PER_TILE // 128)
        def _(i):
            pltpu.sync_copy(r0, acc_sh.at[pl.ds(nbase + i * 128, 128)])

        plsc.subcore_barrier()

        pltpu.make_async_copy(src_h.at[pl.ds(ebase, CPT)], src_v, g0).wait()
        pltpu.make_async_copy(dst_h.at[pl.ds(ebase, CPT)], dst_v, g1).wait()

        def _deg_update(j):
            @pl.when(jnp.where(c == 0, j < CPT // 2, j >= CPT // 2))
            def _():
                @pl.loop(0, 128 // 16)
                def _(k):
                    idx16 = dst_v[j, pl.ds(k * 16, 16)]
                    plsc.addupdate_scatter(deg_v, [idx16], ones16)

        fb = feat_h.at[c]
        bufs = (r0, r1, r2, r3)
        gsem = (g0, g1, g2, g3)
        tsem = (t0, t1, t2, t3)
        NBUF = 4

        for b in range(NBUF):
            pltpu.async_copy(fb.at[src_v.at[b]], bufs[b], gsem[b])

        @pl.loop(0, CPT // NBUF)
        def _(q):
            base = q * NBUF
            for b in range(NBUF):
                j = base + b
                pltpu.make_async_copy(fb.at[src_v.at[j]], bufs[b], gsem[b]).wait()
                pltpu.async_copy(bufs[b], acc_sh.at[dst_v.at[j]], tsem[b],
                                 add=True)
                _deg_update(j)
            for b in range(NBUF):
                j = base + b

                @pl.when(j + NBUF < CPT)
                def _():
                    pltpu.make_async_copy(bufs[b], acc_sh.at[dst_v.at[j]],
                                          tsem[b]).wait()
                    pltpu.async_copy(fb.at[src_v.at[j + NBUF]], bufs[b], gsem[b])

        for b in range(NBUF):
            j = CPT - NBUF + b
            pltpu.make_async_copy(bufs[b], acc_sh.at[dst_v.at[j]],
                                  tsem[b]).wait()

        plsc.subcore_barrier()

        pltpu.sync_copy(acc_sh.at[pl.ds(nbase, ROWS_PER_TILE)],
                        p_out.at[c, pl.ds(nbase, ROWS_PER_TILE)])
        pltpu.sync_copy(deg_v, deg_out.at[c, s])

    return agg(feat2, src2d, dst2d)


def _tc_self(feat, W_self, b_self, b_neigh):
    blk = 2000
    dn = (((1,), (1,)), ((), ()))

    def body(feat_b, ws_b, bs_b, bn_b, out_b):
        out_b[...] = (
            lax.dot_general(feat_b[...], ws_b[...], dn,
                            preferred_element_type=jnp.float32)
            + bs_b[...] + bn_b[...]
        )

    return pl.pallas_call(
        body,
        grid=(N_NODES // blk,),
        in_specs=[
            pl.BlockSpec((blk, D), lambda i: (i, 0)),
            pl.BlockSpec((D, D), lambda i: (0, 0)),
            pl.BlockSpec((1, D), lambda i: (0, 0)),
            pl.BlockSpec((1, D), lambda i: (0, 0)),
        ],
        out_specs=pl.BlockSpec((blk, D), lambda i: (i, 0)),
        out_shape=jax.ShapeDtypeStruct((N_NODES, D), jnp.float32),
    )(feat, W_self, b_self.reshape(1, D), b_neigh.reshape(1, D))


def _tc_neigh(selfpart, p0, p1, deg_t, Wn0, Wn1):
    blk = 2000
    dn = (((1,), (1,)), ((), ()))

    def body(sp_b, p0_b, p1_b, deg_b, wn0_b, wn1_b, out_b):
        deg = jnp.sum(deg_b[...], axis=1, keepdims=True)
        scale = jnp.where(deg > 0.0, 1.0 / jnp.maximum(deg, 1.0), 0.0)
        h0 = p0_b[...] * scale
        h1 = p1_b[...] * scale
        out_b[...] = (
            sp_b[...]
            + lax.dot_general(h0, wn0_b[...], dn,
                              preferred_element_type=jnp.float32)
            + lax.dot_general(h1, wn1_b[...], dn,
                              preferred_element_type=jnp.float32)
        )

    return pl.pallas_call(
        body,
        grid=(N_NODES // blk,),
        in_specs=[
            pl.BlockSpec((blk, D), lambda i: (i, 0)),
            pl.BlockSpec((blk, DH), lambda i: (i, 0)),
            pl.BlockSpec((blk, DH), lambda i: (i, 0)),
            pl.BlockSpec((blk, NW), lambda i: (i, 0)),
            pl.BlockSpec((D, DH), lambda i: (0, 0)),
            pl.BlockSpec((D, DH), lambda i: (0, 0)),
        ],
        out_specs=pl.BlockSpec((blk, D), lambda i: (i, 0)),
        out_shape=jax.ShapeDtypeStruct((N_NODES, D), jnp.float32),
    )(selfpart, p0, p1, deg_t, Wn0, Wn1)


def kernel(feat, edge_index, W_self, b_self, W_neigh, b_neigh):
    src = edge_index[0]
    dst = edge_index[1]
    pad = E_PAD - src.shape[0]

    def _to_tile_chunks(x):
        return x.reshape(CPT, NS, 128).transpose(1, 0, 2).reshape(E_PAD // 128, 128)

    src_p = _to_tile_chunks(jnp.concatenate([src, jnp.zeros((pad,), jnp.int32)]))
    pad_dst = N_NODES + jnp.arange(pad, dtype=jnp.int32) % (N_PAD - N_NODES)
    dst_p = _to_tile_chunks(jnp.concatenate([dst, pad_dst]))

    feat2 = jnp.stack([feat[:, :DH], feat[:, DH:]])
    selfpart = _tc_self(feat, W_self, b_self, b_neigh)
    p, degp = _sc_aggregate(feat2, src_p, dst_p)
    deg_t = degp.reshape(NW, N_PAD).transpose(1, 0)
    return _tc_neigh(selfpart, p[0], p[1], deg_t,
                     W_neigh[:, :DH], W_neigh[:, DH:])


# speedup vs baseline: 1.1184x; 1.0207x over previous
import dataclasses
import functools

import numpy as np

import jax
import jax.numpy as jnp
from jax import lax
from jax.experimental import pallas as pl
from jax.experimental.pallas import tpu as pltpu
from jax.experimental.pallas import tpu_sc as plsc

N_NODES = 10000
D = 128
DH = D // 2
NC, NS = 2, 16
NW = NC * NS
N_PAD = 10240
CPT = 160
E_PAD = NS * CPT * 128
ROWS_PER_TILE = N_PAD // NS


def _sc_compiler_params():
    cp = pltpu.CompilerParams()
    fields = pltpu.CompilerParams.__dataclass_fields__
    if "needs_layout_passes" in fields:
        cp = dataclasses.replace(cp, needs_layout_passes=False)
    if "use_tc_tiling_on_sc" in fields:
        cp = dataclasses.replace(cp, use_tc_tiling_on_sc=False)
    return cp


def _sc_aggregate(feat2, src2d, dst2d):
    mesh = plsc.VectorSubcoreMesh(core_axis_name="c", subcore_axis_name="s")

    @functools.partial(
        pl.kernel,
        mesh=mesh,
        compiler_params=_sc_compiler_params(),
        out_type=(
            jax.ShapeDtypeStruct((NC, N_PAD, DH), jnp.float32),
            jax.ShapeDtypeStruct((NC, NS, N_PAD), jnp.float32),
        ),
        scratch_types=[
            pltpu.VMEM((CPT, 128), jnp.int32),
            pltpu.VMEM((CPT, 128), jnp.int32),
            pltpu.VMEM((128, DH), jnp.float32),
            pltpu.VMEM((128, DH), jnp.float32),
            pltpu.VMEM((128, DH), jnp.float32),
            pltpu.VMEM((128, DH), jnp.float32),
            pltpu.VMEM((N_PAD,), jnp.float32),
            pltpu.VMEM_SHARED((N_PAD, DH), jnp.float32),
            pltpu.SemaphoreType.DMA,
            pltpu.SemaphoreType.DMA,
            pltpu.SemaphoreType.DMA,
            pltpu.SemaphoreType.DMA,
            pltpu.SemaphoreType.DMA,
            pltpu.SemaphoreType.DMA,
            pltpu.SemaphoreType.DMA,
            pltpu.SemaphoreType.DMA,
        ],
    )
    def agg(feat_h, src_h, dst_h, p_out, deg_out,
            src_v, dst_v, r0, r1, r2, r3, deg_v, acc_sh,
            g0, g1, g2, g3, t0, t1, t2, t3):
        c = lax.axis_index("c")
        s = lax.axis_index("s")
        zeros16 = jnp.zeros((16,), jnp.float32)
        ones16 = jnp.ones((16,), jnp.float32)

        ebase = s * CPT
        pltpu.async_copy(src_h.at[pl.ds(ebase, CPT)], src_v, g0)
        pltpu.async_copy(dst_h.at[pl.ds(ebase, CPT)], dst_v, g1)

        @pl.loop(0, N_PAD // 16)
        def _(i):
            deg_v[pl.ds(i * 16, 16)] = zeros16

        @pl.loop(0, 128)
        def _(r):
            @pl.loop(0, DH // 16)
            def _(k):
                r0[r, pl.ds(k * 16, 16)] = zeros16

        nbase = s * ROWS_PER_TILE

        @pl.loop(0, ROWS_PER_TILE // 128)
        def _(i):
            pltpu.sync_copy(r0, acc_sh.at[pl.ds(nbase + i * 128, 128)])

        plsc.subcore_barrier()

        pltpu.make_async_copy(src_h.at[pl.ds(ebase, CPT)], src_v, g0).wait()
        pltpu.make_async_copy(dst_h.at[pl.ds(ebase, CPT)], dst_v, g1).wait()

        def _deg_update(j):
            @pl.when(jnp.where(c == 0, j < CPT // 2, j >= CPT // 2))
            def _():
                @pl.loop(0, 128 // 16)
                def _(k):
                    idx16 = dst_v[j, pl.ds(k * 16, 16)]
                    plsc.addupdate_scatter(deg_v, [idx16], ones16)

        fb = feat_h.at[c]
        bufs = (r0, r1, r2, r3)
        gsem = (g0, g1, g2, g3)
        tsem = (t0, t1, t2, t3)
        NBUF = 4

        for b in range(NBUF):
            pltpu.async_copy(fb.at[src_v.at[b]], bufs[b], gsem[b])

        @pl.loop(0, CPT // NBUF)
        def _(q):
            base = q * NBUF
            for b in range(NBUF):
                j = base + b
                pltpu.make_async_copy(fb.at[src_v.at[j]], bufs[b], gsem[b]).wait()
                pltpu.async_copy(bufs[b], acc_sh.at[dst_v.at[j]], tsem[b],
                                 add=True)
                _deg_update(j)
            for b in range(NBUF):
                j = base + b

                @pl.when(j + NBUF < CPT)
                def _():
                    pltpu.make_async_copy(bufs[b], acc_sh.at[dst_v.at[j]],
                                          tsem[b]).wait()
                    pltpu.async_copy(fb.at[src_v.at[j + NBUF]], bufs[b], gsem[b])

        for b in range(NBUF):
            j = CPT - NBUF + b
            pltpu.make_async_copy(bufs[b], acc_sh.at[dst_v.at[j]],
                                  tsem[b]).wait()

        plsc.subcore_barrier()

        pltpu.sync_copy(acc_sh.at[pl.ds(nbase, ROWS_PER_TILE)],
                        p_out.at[c, pl.ds(nbase, ROWS_PER_TILE)])
        pltpu.sync_copy(deg_v, deg_out.at[c, s])

    return agg(feat2, src2d, dst2d)


def _tc_self(feat, W_self, b_self, b_neigh):
    blk = 2000
    dn = (((1,), (1,)), ((), ()))

    def body(feat_b, ws_b, bs_b, bn_b, out_b):
        out_b[...] = (
            lax.dot_general(feat_b[...], ws_b[...], dn,
                            preferred_element_type=jnp.float32)
            + bs_b[...] + bn_b[...]
        )

    return pl.pallas_call(
        body,
        grid=(N_NODES // blk,),
        in_specs=[
            pl.BlockSpec((blk, D), lambda i: (i, 0)),
            pl.BlockSpec((D, D), lambda i: (0, 0)),
            pl.BlockSpec((1, D), lambda i: (0, 0)),
            pl.BlockSpec((1, D), lambda i: (0, 0)),
        ],
        out_specs=pl.BlockSpec((blk, D), lambda i: (i, 0)),
        out_shape=jax.ShapeDtypeStruct((N_NODES, D), jnp.float32),
    )(feat, W_self, b_self.reshape(1, D), b_neigh.reshape(1, D))


def _tc_neigh(selfpart, p0, p1, deg_t, Wn0, Wn1):
    blk = 2000
    dn = (((1,), (1,)), ((), ()))

    def body(sp_b, p0_b, p1_b, deg_b, wn0_b, wn1_b, out_b):
        deg = jnp.sum(deg_b[...], axis=1, keepdims=True)
        scale = jnp.where(deg > 0.0, 1.0 / jnp.maximum(deg, 1.0), 0.0)
        h0 = p0_b[...] * scale
        h1 = p1_b[...] * scale
        out_b[...] = (
            sp_b[...]
            + lax.dot_general(h0, wn0_b[...], dn,
                              preferred_element_type=jnp.float32)
            + lax.dot_general(h1, wn1_b[...], dn,
                              preferred_element_type=jnp.float32)
        )

    return pl.pallas_call(
        body,
        grid=(N_NODES // blk,),
        in_specs=[
            pl.BlockSpec((blk, D), lambda i: (i, 0)),
            pl.BlockSpec((blk, DH), lambda i: (i, 0)),
            pl.BlockSpec((blk, DH), lambda i: (i, 0)),
            pl.BlockSpec((blk, NW), lambda i: (i, 0)),
            pl.BlockSpec((D, DH), lambda i: (0, 0)),
            pl.BlockSpec((D, DH), lambda i: (0, 0)),
        ],
        out_specs=pl.BlockSpec((blk, D), lambda i: (i, 0)),
        out_shape=jax.ShapeDtypeStruct((N_NODES, D), jnp.float32),
    )(selfpart, p0, p1, deg_t, Wn0, Wn1)


def kernel(feat, edge_index, W_self, b_self, W_neigh, b_neigh):
    pad = E_PAD - edge_index.shape[1]
    pad_block = np.stack([
        np.zeros((pad,), np.int32),
        (N_NODES + np.arange(pad, dtype=np.int32) % (N_PAD - N_NODES)),
    ])
    ei = jnp.concatenate([edge_index, jnp.asarray(pad_block)], axis=1)
    ei = ei.reshape(2, CPT, NS, 128).transpose(0, 2, 1, 3).reshape(
        2, E_PAD // 128, 128)
    src_p = ei[0]
    dst_p = ei[1]

    feat2 = jnp.stack([feat[:, :DH], feat[:, DH:]])
    selfpart = _tc_self(feat, W_self, b_self, b_neigh)
    p, degp = _sc_aggregate(feat2, src_p, dst_p)
    deg_t = degp.reshape(NW, N_PAD).transpose(1, 0)
    return _tc_neigh(selfpart, p[0], p[1], deg_t,
                     W_neigh[:, :DH], W_neigh[:, DH:])
